# baseline (speedup 1.0000x reference)
import functools

import jax
import jax.numpy as jnp
from jax import lax
from jax.experimental import pallas as pl
from jax.experimental.pallas import tpu as pltpu
from jax.experimental.pallas import tpu_sc as plsc

ROWS = 65536
C = 313
LANES = 16
NW = 32
RPW = ROWS // NW
CHUNK_ROWS = 64
GROUPS = CHUNK_ROWS // LANES
NCHUNK = RPW // CHUNK_ROWS
CHUNK_WORDS = CHUNK_ROWS * C
NEG = -3.4e38


def _compute_chunk(ob, bb, acc):
    lanes = lax.iota(jnp.int32, LANES)
    bases = [lanes * C + jnp.int32(g * LANES * C) for g in range(GROUPS)]

    neg = jnp.full((LANES,), NEG, jnp.float32)
    init = ((neg,) * GROUPS, (neg,) * GROUPS, (neg,) * GROUPS)

    def step(c, st):
        bbest, obest, omax = st
        nb, no, nm = [], [], []
        for g in range(GROUPS):
            idx = bases[g] + c
            bv = plsc.load_gather(bb, [idx])
            ov = plsc.load_gather(ob, [idx])
            better = bv > bbest[g]
            nb.append(jnp.where(better, bv, bbest[g]))
            no.append(jnp.where(better, ov, obest[g]))
            nm.append(jnp.maximum(omax[g], ov))
        return (tuple(nb), tuple(no), tuple(nm))

    _, obest, omax = lax.fori_loop(0, C, step, init)
    for g in range(GROUPS):
        acc = acc + (1.0 - obest[g] / omax[g])
    return acc


@functools.partial(
    pl.kernel,
    out_type=jax.ShapeDtypeStruct((NW, LANES), jnp.float32),
    mesh=plsc.VectorSubcoreMesh(core_axis_name="c", subcore_axis_name="s"),
    compiler_params=pltpu.CompilerParams(needs_layout_passes=False),
    scratch_types=[
        pltpu.VMEM((CHUNK_WORDS,), jnp.float32),
        pltpu.VMEM((CHUNK_WORDS,), jnp.float32),
        pltpu.VMEM((CHUNK_WORDS,), jnp.float32),
        pltpu.VMEM((CHUNK_WORDS,), jnp.float32),
        pltpu.VMEM((LANES,), jnp.float32),
        pltpu.SemaphoreType.DMA,
        pltpu.SemaphoreType.DMA,
        pltpu.SemaphoreType.DMA,
        pltpu.SemaphoreType.DMA,
    ],
)
def _sc_loss(orig_hbm, boost_hbm, out_hbm,
             ob0, bb0, ob1, bb1, accbuf, so0, sb0, so1, sb1):
    wid = lax.axis_index("s") * 2 + lax.axis_index("c")
    wbase = wid * (RPW * C)

    def start(ob, bb, so, sb, g):
        off = wbase + g * CHUNK_WORDS
        pltpu.async_copy(orig_hbm.at[pl.ds(off, CHUNK_WORDS)], ob, so)
        pltpu.async_copy(boost_hbm.at[pl.ds(off, CHUNK_WORDS)], bb, sb)

    def wait(ob, bb, so, sb, g):
        off = wbase + g * CHUNK_WORDS
        pltpu.make_async_copy(orig_hbm.at[pl.ds(off, CHUNK_WORDS)], ob, so).wait()
        pltpu.make_async_copy(boost_hbm.at[pl.ds(off, CHUNK_WORDS)], bb, sb).wait()

    start(ob0, bb0, so0, sb0, 0)

    def outer(i, acc):
        g0 = 2 * i
        wait(ob0, bb0, so0, sb0, g0)
        start(ob1, bb1, so1, sb1, g0 + 1)
        acc = _compute_chunk(ob0, bb0, acc)
        wait(ob1, bb1, so1, sb1, g0 + 1)

        @pl.when(i < NCHUNK // 2 - 1)
        def _():
            start(ob0, bb0, so0, sb0, g0 + 2)

        return _compute_chunk(ob1, bb1, acc)

    acc = lax.fori_loop(0, NCHUNK // 2, outer, jnp.zeros((LANES,), jnp.float32))
    accbuf[...] = acc
    pltpu.sync_copy(accbuf, out_hbm.at[wid])


def kernel(original, boosted):
    orig = original.reshape(-1)
    boost = boosted.reshape(-1)
    partials = _sc_loss(orig, boost)
    return jnp.sum(partials)


# speedup vs baseline: 1.0039x; 1.0039x over previous
import functools

import jax
import jax.numpy as jnp
from jax import lax
from jax.experimental import pallas as pl
from jax.experimental.pallas import tpu as pltpu
from jax.experimental.pallas import tpu_sc as plsc

ROWS = 65536
C = 313
LANES = 16
NW = 32
RPW = ROWS // NW
CHUNK_ROWS = 64
GROUPS = CHUNK_ROWS // LANES
NCHUNK = RPW // CHUNK_ROWS
CHUNK_WORDS = CHUNK_ROWS * C
NEG = -3.4e38


def _compute_chunk(ob, bb, acc):
    lanes = lax.iota(jnp.int32, LANES)
    bases = [lanes * C + jnp.int32(g * LANES * C) for g in range(GROUPS)]

    neg = jnp.full((LANES,), NEG, jnp.float32)
    init = ((neg,) * GROUPS, (neg,) * GROUPS, (neg,) * GROUPS)

    @plsc.parallel_loop(0, C, carry=init, unroll=4)
    def step(c, st):
        bbest, obest, omax = st
        nb, no, nm = [], [], []
        for g in range(GROUPS):
            idx = bases[g] + c
            bv = plsc.load_gather(bb, [idx])
            ov = plsc.load_gather(ob, [idx])
            better = bv > bbest[g]
            nb.append(jnp.where(better, bv, bbest[g]))
            no.append(jnp.where(better, ov, obest[g]))
            nm.append(jnp.maximum(omax[g], ov))
        return (tuple(nb), tuple(no), tuple(nm))

    _, obest, omax = step
    for g in range(GROUPS):
        acc = acc + (1.0 - obest[g] / omax[g])
    return acc


@functools.partial(
    pl.kernel,
    out_type=jax.ShapeDtypeStruct((NW, LANES), jnp.float32),
    mesh=plsc.VectorSubcoreMesh(core_axis_name="c", subcore_axis_name="s"),
    compiler_params=pltpu.CompilerParams(needs_layout_passes=False),
    scratch_types=[
        pltpu.VMEM((CHUNK_WORDS,), jnp.float32),
        pltpu.VMEM((CHUNK_WORDS,), jnp.float32),
        pltpu.VMEM((CHUNK_WORDS,), jnp.float32),
        pltpu.VMEM((CHUNK_WORDS,), jnp.float32),
        pltpu.VMEM((LANES,), jnp.float32),
        pltpu.SemaphoreType.DMA,
        pltpu.SemaphoreType.DMA,
        pltpu.SemaphoreType.DMA,
        pltpu.SemaphoreType.DMA,
    ],
)
def _sc_loss(orig_hbm, boost_hbm, out_hbm,
             ob0, bb0, ob1, bb1, accbuf, so0, sb0, so1, sb1):
    wid = lax.axis_index("s") * 2 + lax.axis_index("c")
    wbase = wid * (RPW * C)

    def start(ob, bb, so, sb, g):
        off = wbase + g * CHUNK_WORDS
        pltpu.async_copy(orig_hbm.at[pl.ds(off, CHUNK_WORDS)], ob, so)
        pltpu.async_copy(boost_hbm.at[pl.ds(off, CHUNK_WORDS)], bb, sb)

    def wait(ob, bb, so, sb, g):
        off = wbase + g * CHUNK_WORDS
        pltpu.make_async_copy(orig_hbm.at[pl.ds(off, CHUNK_WORDS)], ob, so).wait()
        pltpu.make_async_copy(boost_hbm.at[pl.ds(off, CHUNK_WORDS)], bb, sb).wait()

    start(ob0, bb0, so0, sb0, 0)

    def outer(i, acc):
        g0 = 2 * i
        wait(ob0, bb0, so0, sb0, g0)
        start(ob1, bb1, so1, sb1, g0 + 1)
        acc = _compute_chunk(ob0, bb0, acc)
        wait(ob1, bb1, so1, sb1, g0 + 1)

        @pl.when(i < NCHUNK // 2 - 1)
        def _():
            start(ob0, bb0, so0, sb0, g0 + 2)

        return _compute_chunk(ob1, bb1, acc)

    acc = lax.fori_loop(0, NCHUNK // 2, outer, jnp.zeros((LANES,), jnp.float32))
    accbuf[...] = acc
    pltpu.sync_copy(accbuf, out_hbm.at[wid])


def kernel(original, boosted):
    orig = original.reshape(-1)
    boost = boosted.reshape(-1)
    partials = _sc_loss(orig, boost)
    return jnp.sum(partials)
